# baseline (speedup 1.0000x reference)
import functools

import jax
import jax.numpy as jnp
from jax import lax
from jax.experimental import pallas as pl
from jax.experimental.pallas import tpu as pltpu
from jax.experimental.pallas import tpu_sc as plsc

BATCH = 8
D_IN = 768
D_CB = 64
T = 1024
K = 8192
CB_BLK = 2048
NCB = K // CB_BLK
NTOK = BATCH * T


def _project_body(z_ref, wv_ref, g_ref, b_ref, ze_out):
    v = wv_ref[:, :]
    norm = jnp.sqrt(jnp.sum(v * v, axis=1, keepdims=True))
    W = (g_ref[:, :] * v) / norm
    ze = jnp.dot(W, z_ref[0], preferred_element_type=jnp.float32)
    ze_out[0] = ze + b_ref[:, :]


def _project(z, W_in_v, g_in, b_in):
    return pl.pallas_call(
        _project_body,
        grid=(BATCH,),
        in_specs=[
            pl.BlockSpec((1, D_IN, T), lambda b: (b, 0, 0)),
            pl.BlockSpec((D_CB, D_IN), lambda b: (0, 0)),
            pl.BlockSpec((D_CB, 1), lambda b: (0, 0)),
            pl.BlockSpec((D_CB, 1), lambda b: (0, 0)),
        ],
        out_specs=pl.BlockSpec((1, D_CB, T), lambda b: (b, 0, 0)),
        out_shape=jax.ShapeDtypeStruct((BATCH, D_CB, T), jnp.float32),
    )(z, W_in_v, g_in.reshape(D_CB, 1), b_in.reshape(D_CB, 1))


def _rbf16(x):
    bu = lax.bitcast_convert_type(x, jnp.uint32)
    lsb = (bu >> 16) & jnp.uint32(1)
    r = (bu + jnp.uint32(0x7FFF) + lsb) & jnp.uint32(0xFFFF0000)
    return lax.bitcast_convert_type(r, jnp.float32)


def _argmax_body(e_ref, e2_ref, c_ref, c2_ref, o_ref, val_s, idx_s):
    j = pl.program_id(1)

    @pl.when(j == 0)
    def _():
        val_s[...] = jnp.full((T, 1), -jnp.inf, jnp.float32)
        idx_s[...] = jnp.zeros((T, 1), jnp.int32)

    s = lax.dot_general(e_ref[...], c_ref[...], (((1,), (1,)), ((), ())),
                        preferred_element_type=jnp.float32)
    dist = e2_ref[...] - 2.0 * s + c2_ref[...]
    nd = -dist
    lmax = jnp.max(nd, axis=1, keepdims=True)
    larg = jnp.argmax(nd, axis=1).astype(jnp.int32)[:, None] + j * CB_BLK
    upd = lmax > val_s[...]
    idx_s[...] = jnp.where(upd, larg, idx_s[...])
    val_s[...] = jnp.where(upd, _rbf16(lmax), val_s[...])

    @pl.when(j == NCB - 1)
    def _():
        o_ref[...] = idx_s[...]


def _argmax(encn, e2, cbn, cn2):
    return pl.pallas_call(
        _argmax_body,
        grid=(BATCH, NCB),
        in_specs=[
            pl.BlockSpec((T, D_CB), lambda i, j: (i, 0)),
            pl.BlockSpec((T, 1), lambda i, j: (i, 0)),
            pl.BlockSpec((CB_BLK, D_CB), lambda i, j: (j, 0)),
            pl.BlockSpec((1, CB_BLK), lambda i, j: (0, j)),
        ],
        out_specs=pl.BlockSpec((T, 1), lambda i, j: (i, 0)),
        out_shape=jax.ShapeDtypeStruct((NTOK, 1), jnp.int32),
        scratch_shapes=[pltpu.VMEM((T, 1), jnp.float32),
                        pltpu.VMEM((T, 1), jnp.int32)],
    )(encn, e2, cbn, cn2)


def _sc_gather(codebook, indices_flat):
    info = plsc.get_sparse_core_info()
    NC, NS = info.num_cores, info.num_subcores
    NW = NC * NS
    per_w = NTOK // NW
    CH = 128
    nch = per_w // CH
    idx2d = indices_flat.reshape(NTOK // CH, CH)
    mesh = plsc.VectorSubcoreMesh(core_axis_name="c", subcore_axis_name="s")

    @functools.partial(
        pl.kernel,
        out_type=jax.ShapeDtypeStruct((NTOK // CH, CH, D_CB), jnp.float32),
        mesh=mesh,
        compiler_params=pltpu.CompilerParams(use_tc_tiling_on_sc=False),
        scratch_types=[
            pltpu.VMEM((nch, CH), jnp.int32),
            pltpu.VMEM((nch, CH, D_CB), jnp.float32),
            pltpu.SemaphoreType.DMA,
        ],
    )
    def gather_k(table_hbm, idx_hbm, out_hbm, idx_v, rows_v, sem):
        wid = lax.axis_index("s") * NC + lax.axis_index("c")
        base = wid * nch
        pltpu.sync_copy(idx_hbm.at[pl.ds(base, nch)], idx_v)
        cps = [pltpu.async_copy(table_hbm.at[idx_v.at[j]], rows_v.at[j], sem)
               for j in range(nch)]
        for cp in cps:
            cp.wait()
        pltpu.sync_copy(rows_v, out_hbm.at[pl.ds(base, nch)])

    rows = gather_k(codebook, idx2d)
    return rows.reshape(NTOK, D_CB)


def _decode_body(zq_ref, ze_ref, wv_ref, g_ref, b_ref,
                 out_ref, com_ref, cbl_ref):
    v = wv_ref[:, :]
    norm = jnp.sqrt(jnp.sum(v * v, axis=1, keepdims=True))
    W = (g_ref[:, :] * v) / norm
    zq = zq_ref[0]
    out = lax.dot_general(W, zq, (((1,), (1,)), ((), ())),
                          preferred_element_type=jnp.float32)
    out_ref[0] = out + b_ref[:, :]
    ze = ze_ref[0]
    m = jnp.dot(ze, zq, preferred_element_type=jnp.float32,
                precision=lax.Precision.HIGHEST)
    eye = (lax.broadcasted_iota(jnp.int32, (D_CB, D_CB), 0)
           == lax.broadcasted_iota(jnp.int32, (D_CB, D_CB), 1))
    cross = jnp.sum(jnp.where(eye, m, 0.0))
    sq = jnp.sum(ze * ze) + jnp.sum(zq * zq) - 2.0 * cross
    mse = sq / float(D_CB * T)
    com_ref[0] = jnp.full((8, 128), mse * 0.005, jnp.float32)
    cbl_ref[0] = jnp.full((8, 128), mse * 1.0, jnp.float32)


def _decode(z_q, z_e, W_out_v, g_out, b_out):
    return pl.pallas_call(
        _decode_body,
        grid=(BATCH,),
        in_specs=[
            pl.BlockSpec((1, T, D_CB), lambda b: (b, 0, 0)),
            pl.BlockSpec((1, D_CB, T), lambda b: (b, 0, 0)),
            pl.BlockSpec((D_IN, D_CB), lambda b: (0, 0)),
            pl.BlockSpec((D_IN, 1), lambda b: (0, 0)),
            pl.BlockSpec((D_IN, 1), lambda b: (0, 0)),
        ],
        out_specs=[
            pl.BlockSpec((1, D_IN, T), lambda b: (b, 0, 0)),
            pl.BlockSpec((1, 8, 128), lambda b: (b, 0, 0)),
            pl.BlockSpec((1, 8, 128), lambda b: (b, 0, 0)),
        ],
        out_shape=[
            jax.ShapeDtypeStruct((BATCH, D_IN, T), jnp.float32),
            jax.ShapeDtypeStruct((BATCH, 8, 128), jnp.float32),
            jax.ShapeDtypeStruct((BATCH, 8, 128), jnp.float32),
        ],
    )(z_q, z_e, W_out_v, g_out.reshape(D_IN, 1), b_out.reshape(D_IN, 1))


def kernel(z, W_in_v, g_in, b_in, W_out_v, g_out, b_out, codebook):
    z_e = _project(z, W_in_v, g_in, b_in)
    enc = jnp.transpose(z_e, (0, 2, 1)).reshape(-1, D_CB)
    n = jnp.sqrt(jnp.sum(enc * enc, axis=1, keepdims=True))
    encn = enc / jnp.maximum(n, 1e-12)
    e2 = jnp.sum(encn * encn, axis=1, keepdims=True)
    cnorm = jnp.sqrt(jnp.sum(codebook * codebook, axis=1, keepdims=True))
    cbn = codebook / jnp.maximum(cnorm, 1e-12)
    cn2 = jnp.sum(cbn * cbn, axis=1)[None, :]
    idx = _argmax(encn, e2, cbn, cn2)
    indices = idx.reshape(BATCH, T)
    z_q_flat = _sc_gather(codebook, idx.reshape(-1))
    z_q = z_q_flat.reshape(BATCH, T, D_CB)
    z_q_out, com, cbl = _decode(z_q, z_e, W_out_v, g_out, b_out)
    commit_loss = com[:, 0, 0]
    codebook_loss = cbl[:, 0, 0]
    return z_q_out, commit_loss, codebook_loss, indices, z_e


# speedup vs baseline: 1.4068x; 1.4068x over previous
import functools

import jax
import jax.numpy as jnp
from jax import lax
from jax.experimental import pallas as pl
from jax.experimental.pallas import tpu as pltpu
from jax.experimental.pallas import tpu_sc as plsc

BATCH = 8
D_IN = 768
D_CB = 64
T = 1024
K = 8192
CB_BLK = 2048
NCB = K // CB_BLK
NTOK = BATCH * T


def _project_body(z_ref, wv_ref, g_ref, b_ref, ze_out):
    v = wv_ref[:, :]
    norm = jnp.sqrt(jnp.sum(v * v, axis=1, keepdims=True))
    W = (g_ref[:, :] * v) / norm
    ze = jnp.dot(W, z_ref[0], preferred_element_type=jnp.float32)
    ze_out[0] = ze + b_ref[:, :]


def _project(z, W_in_v, g_in, b_in):
    return pl.pallas_call(
        _project_body,
        grid=(BATCH,),
        in_specs=[
            pl.BlockSpec((1, D_IN, T), lambda b: (b, 0, 0)),
            pl.BlockSpec((D_CB, D_IN), lambda b: (0, 0)),
            pl.BlockSpec((D_CB, 1), lambda b: (0, 0)),
            pl.BlockSpec((D_CB, 1), lambda b: (0, 0)),
        ],
        out_specs=pl.BlockSpec((1, D_CB, T), lambda b: (b, 0, 0)),
        out_shape=jax.ShapeDtypeStruct((BATCH, D_CB, T), jnp.float32),
    )(z, W_in_v, g_in.reshape(D_CB, 1), b_in.reshape(D_CB, 1))


def _rbf16(x):
    bu = lax.bitcast_convert_type(x, jnp.uint32)
    lsb = (bu >> 16) & jnp.uint32(1)
    r = (bu + jnp.uint32(0x7FFF) + lsb) & jnp.uint32(0xFFFF0000)
    return lax.bitcast_convert_type(r, jnp.float32)


def _argmax_body(e_ref, e2_ref, c_ref, c2_ref, o_ref, val_s, idx_s):
    j = pl.program_id(1)

    @pl.when(j == 0)
    def _():
        val_s[...] = jnp.full((1, T), jnp.inf, jnp.float32)
        idx_s[...] = jnp.zeros((1, T), jnp.int32)

    s = lax.dot_general(c_ref[...], e_ref[0], (((0,), (0,)), ((), ())),
                        preferred_element_type=jnp.float32)
    dist = e2_ref[0] - 2.0 * s + c2_ref[...]
    lmin = jnp.min(dist, axis=0, keepdims=True)
    larg = jnp.argmin(dist, axis=0).astype(jnp.int32)[None, :] + j * CB_BLK
    upd = lmin < val_s[...]
    idx_s[...] = jnp.where(upd, larg, idx_s[...])
    val_s[...] = jnp.where(upd, _rbf16(lmin), val_s[...])

    @pl.when(j == NCB - 1)
    def _():
        o_ref[0] = idx_s[...]


def _argmax(encn3, e2, cbn_t, cn2c):
    return pl.pallas_call(
        _argmax_body,
        grid=(BATCH, NCB),
        in_specs=[
            pl.BlockSpec((1, D_CB, T), lambda i, j: (i, 0, 0)),
            pl.BlockSpec((1, 1, T), lambda i, j: (i, 0, 0)),
            pl.BlockSpec((D_CB, CB_BLK), lambda i, j: (0, j)),
            pl.BlockSpec((CB_BLK, 1), lambda i, j: (j, 0)),
        ],
        out_specs=pl.BlockSpec((1, 1, T), lambda i, j: (i, 0, 0)),
        out_shape=jax.ShapeDtypeStruct((BATCH, 1, T), jnp.int32),
        scratch_shapes=[pltpu.VMEM((1, T), jnp.float32),
                        pltpu.VMEM((1, T), jnp.int32)],
    )(encn3, e2, cbn_t, cn2c)


def _sc_gather(codebook, indices_flat):
    info = plsc.get_sparse_core_info()
    NC, NS = info.num_cores, info.num_subcores
    NW = NC * NS
    per_w = NTOK // NW
    CH = 128
    nch = per_w // CH
    idx2d = indices_flat.reshape(NTOK // CH, CH)
    mesh = plsc.VectorSubcoreMesh(core_axis_name="c", subcore_axis_name="s")

    @functools.partial(
        pl.kernel,
        out_type=jax.ShapeDtypeStruct((NTOK // CH, CH, D_CB), jnp.float32),
        mesh=mesh,
        compiler_params=pltpu.CompilerParams(use_tc_tiling_on_sc=False),
        scratch_types=[
            pltpu.VMEM((nch, CH), jnp.int32),
            pltpu.VMEM((nch, CH, D_CB), jnp.float32),
            pltpu.SemaphoreType.DMA,
        ],
    )
    def gather_k(table_hbm, idx_hbm, out_hbm, idx_v, rows_v, sem):
        wid = lax.axis_index("s") * NC + lax.axis_index("c")
        base = wid * nch
        pltpu.sync_copy(idx_hbm.at[pl.ds(base, nch)], idx_v)
        cps = [pltpu.async_copy(table_hbm.at[idx_v.at[j]], rows_v.at[j], sem)
               for j in range(nch)]
        for cp in cps:
            cp.wait()
        pltpu.sync_copy(rows_v, out_hbm.at[pl.ds(base, nch)])

    rows = gather_k(codebook, idx2d)
    return rows.reshape(NTOK, D_CB)


def _decode_body(zq_ref, ze_ref, wv_ref, g_ref, b_ref,
                 out_ref, com_ref, cbl_ref):
    v = wv_ref[:, :]
    norm = jnp.sqrt(jnp.sum(v * v, axis=1, keepdims=True))
    W = (g_ref[:, :] * v) / norm
    zq = zq_ref[0]
    out = lax.dot_general(W, zq, (((1,), (1,)), ((), ())),
                          preferred_element_type=jnp.float32)
    out_ref[0] = out + b_ref[:, :]
    ze = ze_ref[0]
    m = jnp.dot(ze, zq, preferred_element_type=jnp.float32,
                precision=lax.Precision.HIGHEST)
    eye = (lax.broadcasted_iota(jnp.int32, (D_CB, D_CB), 0)
           == lax.broadcasted_iota(jnp.int32, (D_CB, D_CB), 1))
    cross = jnp.sum(jnp.where(eye, m, 0.0))
    sq = jnp.sum(ze * ze) + jnp.sum(zq * zq) - 2.0 * cross
    mse = sq / float(D_CB * T)
    com_ref[0] = jnp.full((8, 128), mse * 0.005, jnp.float32)
    cbl_ref[0] = jnp.full((8, 128), mse * 1.0, jnp.float32)


def _decode(z_q, z_e, W_out_v, g_out, b_out):
    return pl.pallas_call(
        _decode_body,
        grid=(BATCH,),
        in_specs=[
            pl.BlockSpec((1, T, D_CB), lambda b: (b, 0, 0)),
            pl.BlockSpec((1, D_CB, T), lambda b: (b, 0, 0)),
            pl.BlockSpec((D_IN, D_CB), lambda b: (0, 0)),
            pl.BlockSpec((D_IN, 1), lambda b: (0, 0)),
            pl.BlockSpec((D_IN, 1), lambda b: (0, 0)),
        ],
        out_specs=[
            pl.BlockSpec((1, D_IN, T), lambda b: (b, 0, 0)),
            pl.BlockSpec((1, 8, 128), lambda b: (b, 0, 0)),
            pl.BlockSpec((1, 8, 128), lambda b: (b, 0, 0)),
        ],
        out_shape=[
            jax.ShapeDtypeStruct((BATCH, D_IN, T), jnp.float32),
            jax.ShapeDtypeStruct((BATCH, 8, 128), jnp.float32),
            jax.ShapeDtypeStruct((BATCH, 8, 128), jnp.float32),
        ],
    )(z_q, z_e, W_out_v, g_out.reshape(D_IN, 1), b_out.reshape(D_IN, 1))


def kernel(z, W_in_v, g_in, b_in, W_out_v, g_out, b_out, codebook):
    z_e = _project(z, W_in_v, g_in, b_in)
    n3 = jnp.sqrt(jnp.sum(z_e * z_e, axis=1, keepdims=True))
    encn3 = z_e / jnp.maximum(n3, 1e-12)
    e2 = jnp.sum(encn3 * encn3, axis=1, keepdims=True)
    cnorm = jnp.sqrt(jnp.sum(codebook * codebook, axis=1, keepdims=True))
    cbn = codebook / jnp.maximum(cnorm, 1e-12)
    cn2c = jnp.sum(cbn * cbn, axis=1, keepdims=True)
    idx = _argmax(encn3, e2, cbn.T, cn2c)
    indices = idx.reshape(BATCH, T)
    z_q_flat = _sc_gather(codebook, idx.reshape(-1))
    z_q = z_q_flat.reshape(BATCH, T, D_CB)
    z_q_out, com, cbl = _decode(z_q, z_e, W_out_v, g_out, b_out)
    commit_loss = com[:, 0, 0]
    codebook_loss = cbl[:, 0, 0]
    return z_q_out, commit_loss, codebook_loss, indices, z_e
